# baseline (speedup 1.0000x reference)
import functools

import jax
import jax.numpy as jnp
from jax import lax
from jax.experimental import pallas as pl
from jax.experimental.pallas import tpu as pltpu
from jax.experimental.pallas import tpu_sc as plsc

N = 10000
E = 320000
D = 128

NSUB = 16
CH = 96
NCHT = 212
EPAD = NSUB * NCHT * CH
RP = 632
NR = 4


_BLK = 2000


def _dense_body(mean_ref, var_ref, wm_ref, bm_ref, wv_ref, bv_ref, out_ref):
    m = jnp.dot(mean_ref[...], wm_ref[...], preferred_element_type=jnp.float32)
    m = m + bm_ref[...]
    m = jnp.where(m > 0, m, jnp.exp(jnp.minimum(m, 0.0)) - 1.0)
    v = jnp.dot(var_ref[...], wv_ref[...], preferred_element_type=jnp.float32)
    v = v + bv_ref[...]
    v = jnp.maximum(v, 0.0) + 1e-6
    att = jnp.exp(-v)
    out_ref[0] = m * att
    out_ref[1] = v * att * att


def _dense(mean, var, wm, bm, wv, bv):
    grid = (N // _BLK,)
    return pl.pallas_call(
        _dense_body,
        grid=grid,
        in_specs=[
            pl.BlockSpec((_BLK, D), lambda i: (i, 0)),
            pl.BlockSpec((_BLK, D), lambda i: (i, 0)),
            pl.BlockSpec((D, D), lambda i: (0, 0)),
            pl.BlockSpec((1, D), lambda i: (0, 0)),
            pl.BlockSpec((D, D), lambda i: (0, 0)),
            pl.BlockSpec((1, D), lambda i: (0, 0)),
        ],
        out_specs=pl.BlockSpec((2, _BLK, D), lambda i: (0, i, 0)),
        out_shape=jax.ShapeDtypeStruct((2, N, D), jnp.float32),
    )(mean, var, wm, bm, wv, bv)


def _agg_body(x_hbm, row_hbm, col_hbm, w_hbm, out_hbm, acc, *scr):
    colbuf = scr[0:NR]
    rowbuf = scr[NR:2 * NR]
    wbuf = scr[2 * NR:3 * NR]
    gbuf = scr[3 * NR:4 * NR]
    o = 4 * NR
    colsem = scr[o:o + NR]
    rowsem = scr[o + NR:o + 2 * NR]
    wsem = scr[o + 2 * NR:o + 3 * NR]
    gsem = scr[o + 3 * NR:o + 4 * NR]
    ssem = scr[o + 4 * NR:o + 5 * NR]

    c = lax.axis_index("c")
    s = lax.axis_index("s")
    cN = c * N
    cE = c * EPAD
    z16 = jnp.zeros((16,), jnp.float32)
    t0 = s * NCHT
    r0 = s * RP
    g0 = gbuf[0]

    def start_cw(i, e):
        off = (t0 + i) * CH
        pltpu.async_copy(col_hbm.at[pl.ds(off, CH)], colbuf[e], colsem[e])
        pltpu.async_copy(w_hbm.at[pl.ds(cE + off, CH)], wbuf[e], wsem[e])

    def start_row(i, e):
        off = (t0 + i) * CH
        pltpu.async_copy(row_hbm.at[pl.ds(off, CH)], rowbuf[e], rowsem[e])

    def start_gather(i, e, b):
        off = (t0 + i) * CH
        pltpu.make_async_copy(col_hbm.at[pl.ds(off, CH)], colbuf[e],
                              colsem[e]).wait()
        for k in range(CH // 16):
            colbuf[e][pl.ds(k * 16, 16)] = colbuf[e][pl.ds(k * 16, 16)] + cN
        pltpu.async_copy(x_hbm.at[colbuf[e]], gbuf[b], gsem[b])

    def wait_scatter(b):
        pltpu.make_async_copy(gbuf[b], acc.at[rowbuf[0]], ssem[b]).wait()

    for i in range(3):
        start_cw(i, i)
    for i in range(2):
        start_row(i, i)

    def zbody(r, carry):
        for k in range(D // 16):
            g0[r, pl.ds(k * 16, 16)] = z16
        return carry

    lax.fori_loop(0, CH, zbody, 0)

    def acc_zero(nrows):
        for j in range(nrows // CH):
            pltpu.sync_copy(g0.at[pl.ds(0, CH)],
                            acc.at[pl.ds(r0 + j * CH, CH)])
        rem = nrows - (nrows // CH) * CH
        pltpu.sync_copy(g0.at[pl.ds(0, rem)],
                        acc.at[pl.ds(r0 + (nrows // CH) * CH, rem)])

    @pl.when(s < NSUB - 1)
    def _():
        acc_zero(RP)

    @pl.when(s == NSUB - 1)
    def _():
        acc_zero(N - (NSUB - 1) * RP)

    start_gather(0, 0, 0)
    start_gather(1, 1, 1)
    plsc.subcore_barrier()

    def scale(b, e):
        gb = gbuf[b]
        wbf = wbuf[e]

        def bbody(b16, carry):
            wv = wbf[pl.ds(b16 * 16, 16)]
            for l in range(16):
                ed = b16 * 16 + l
                wl = wv[l]
                for k in range(D // 16):
                    sl = gb[ed, pl.ds(k * 16, 16)]
                    gb[ed, pl.ds(k * 16, 16)] = sl * wl
            return carry

        lax.fori_loop(0, CH // 16, bbody, 0)

    def slot(i, j):
        j2 = (j + 2) % NR
        j3 = (j + 3) % NR

        @pl.when(i + 3 < NCHT)
        def _():
            start_cw(i + 3, j3)

        pltpu.make_async_copy(x_hbm.at[colbuf[j]], gbuf[j], gsem[j]).wait()
        pltpu.make_async_copy(w_hbm.at[pl.ds(0, CH)], wbuf[j], wsem[j]).wait()

        @pl.when(i + 2 < NCHT)
        def _():
            @pl.when(i >= 2)
            def _():
                wait_scatter(j2)

            start_row(i + 2, j2)
            start_gather(i + 2, j2, j2)

        scale(j, j)

        pltpu.make_async_copy(row_hbm.at[pl.ds(0, CH)], rowbuf[j],
                              rowsem[j]).wait()
        pltpu.async_copy(gbuf[j], acc.at[rowbuf[j]], ssem[j], add=True)

    def lbody(it, carry):
        for j in range(NR):
            slot(it * NR + j, j)
        return carry

    lax.fori_loop(0, NCHT // NR, lbody, 0)
    for b in range(NR):
        wait_scatter(b)
    plsc.subcore_barrier()

    def out_copies(nrows):
        for j in range(nrows // CH):
            pltpu.sync_copy(acc.at[pl.ds(r0 + j * CH, CH)],
                            g0.at[pl.ds(0, CH)])
            pltpu.sync_copy(g0.at[pl.ds(0, CH)],
                            out_hbm.at[pl.ds(cN + r0 + j * CH, CH)])
        rem = nrows - (nrows // CH) * CH
        pltpu.sync_copy(acc.at[pl.ds(r0 + (nrows // CH) * CH, rem)],
                        g0.at[pl.ds(0, rem)])
        pltpu.sync_copy(g0.at[pl.ds(0, rem)],
                        out_hbm.at[pl.ds(cN + r0 + (nrows // CH) * CH, rem)])

    @pl.when(s < NSUB - 1)
    def _():
        out_copies(RP)

    @pl.when(s == NSUB - 1)
    def _():
        out_copies(N - (NSUB - 1) * RP)


def _agg(x_all, row, col, w_all):
    mesh = plsc.VectorSubcoreMesh(core_axis_name="c", subcore_axis_name="s")
    f = functools.partial(
        pl.kernel,
        out_type=jax.ShapeDtypeStruct((2 * N, D), jnp.float32),
        mesh=mesh,
        compiler_params=pltpu.CompilerParams(needs_layout_passes=False),
        scratch_types=(
            [pltpu.VMEM_SHARED((N, D), jnp.float32)]
            + [pltpu.VMEM((CH,), jnp.int32) for _ in range(NR)]
            + [pltpu.VMEM((CH,), jnp.int32) for _ in range(NR)]
            + [pltpu.VMEM((CH,), jnp.float32) for _ in range(NR)]
            + [pltpu.VMEM((CH, D), jnp.float32) for _ in range(NR)]
            + [pltpu.SemaphoreType.DMA for _ in range(5 * NR)]
        ),
    )(_agg_body)
    return f(x_all, row, col, w_all)


def kernel(mean, var, edge_index, edge_weight0, edge_weight1,
           W_mean, b_mean, W_var, b_var):
    xs = _dense(mean, var, W_mean, b_mean.reshape(1, D),
                W_var, b_var.reshape(1, D))
    x_all = xs.reshape(2 * N, D)
    pad = EPAD - E
    row = jnp.pad(edge_index[0], (0, pad))
    col = jnp.pad(edge_index[1], (0, pad))
    w_all = jnp.concatenate([
        jnp.pad(edge_weight0, (0, pad)),
        jnp.pad(edge_weight1, (0, pad)),
    ])
    out = _agg(x_all, row, col, w_all)
    return out[:N], out[N:]


# speedup vs baseline: 1.1716x; 1.1716x over previous
import functools

import jax
import jax.numpy as jnp
from jax import lax
from jax.experimental import pallas as pl
from jax.experimental.pallas import tpu as pltpu
from jax.experimental.pallas import tpu_sc as plsc

N = 10000
E = 320000
D = 128

NSUB = 16
CH = 112
NCHT = 180
EPAD = NSUB * NCHT * CH
NCH = EPAD // CH
RP = 632
NB = 3


_BLK = 2000


def _dense_body(mean_ref, var_ref, wm_ref, bm_ref, wv_ref, bv_ref, out_ref):
    m = jnp.dot(mean_ref[...], wm_ref[...], preferred_element_type=jnp.float32)
    m = m + bm_ref[...]
    m = jnp.where(m > 0, m, jnp.exp(jnp.minimum(m, 0.0)) - 1.0)
    v = jnp.dot(var_ref[...], wv_ref[...], preferred_element_type=jnp.float32)
    v = v + bv_ref[...]
    v = jnp.maximum(v, 0.0) + 1e-6
    att = jnp.exp(-v)
    out_ref[0] = m * att
    out_ref[1] = v * att * att


def _dense(mean, var, wm, bm, wv, bv):
    grid = (N // _BLK,)
    return pl.pallas_call(
        _dense_body,
        grid=grid,
        in_specs=[
            pl.BlockSpec((_BLK, D), lambda i: (i, 0)),
            pl.BlockSpec((_BLK, D), lambda i: (i, 0)),
            pl.BlockSpec((D, D), lambda i: (0, 0)),
            pl.BlockSpec((1, D), lambda i: (0, 0)),
            pl.BlockSpec((D, D), lambda i: (0, 0)),
            pl.BlockSpec((1, D), lambda i: (0, 0)),
        ],
        out_specs=pl.BlockSpec((2, _BLK, D), lambda i: (0, i, 0)),
        out_shape=jax.ShapeDtypeStruct((2, N, D), jnp.float32),
    )(mean, var, wm, bm, wv, bv)


def _agg_body(x_hbm, ed_hbm, out_hbm, acc, *scr):
    pbuf = scr[0:NB]
    rowb = scr[NB:2 * NB]
    gbuf = scr[2 * NB:3 * NB]
    o = 3 * NB
    psem = scr[o:o + NB]
    gsem = scr[o + NB:o + 2 * NB]
    ssem = scr[o + 2 * NB:o + 3 * NB]

    c = lax.axis_index("c")
    s = lax.axis_index("s")
    cN = c * N
    z16 = jnp.zeros((16,), jnp.float32)
    t0 = s * NCHT
    r0 = s * RP
    g0 = gbuf[0]
    NL = CH // 16

    def start_pair(i, pq):
        pltpu.async_copy(ed_hbm.at[pl.ds(4 * (t0 + i), 8)], pbuf[pq],
                         psem[pq])

    def wait_pair(i, pq):
        pltpu.make_async_copy(ed_hbm.at[pl.ds(4 * (t0 + i), 8)], pbuf[pq],
                              psem[pq]).wait()

    def start_gather(p, pq, b):
        pb = pbuf[pq]
        for k in range(NL):
            pb[4 * p, pl.ds(k * 16, 16)] = pb[4 * p, pl.ds(k * 16, 16)] + cN
        pltpu.async_copy(x_hbm.at[pb.at[4 * p]], gbuf[b], gsem[b])

    def wait_gather(b):
        pltpu.make_async_copy(x_hbm.at[rowb[0]], gbuf[b], gsem[b]).wait()

    def wait_scatter(b):
        pltpu.make_async_copy(gbuf[b], acc.at[rowb[b]], ssem[b]).wait()

    start_pair(0, 0)
    start_pair(2, 1)

    def zbody(r, carry):
        for k in range(D // 16):
            g0[r, pl.ds(k * 16, 16)] = z16
        return carry

    lax.fori_loop(0, CH, zbody, 0)

    def acc_zero(nrows):
        for j in range(nrows // CH):
            pltpu.sync_copy(g0.at[pl.ds(0, CH)],
                            acc.at[pl.ds(r0 + j * CH, CH)])
        rem = nrows - (nrows // CH) * CH
        pltpu.sync_copy(g0.at[pl.ds(0, rem)],
                        acc.at[pl.ds(r0 + (nrows // CH) * CH, rem)])

    @pl.when(s < NSUB - 1)
    def _():
        acc_zero(RP)

    @pl.when(s == NSUB - 1)
    def _():
        acc_zero(N - (NSUB - 1) * RP)

    wait_pair(0, 0)
    start_gather(0, 0, 0)
    start_gather(1, 0, 1)
    plsc.subcore_barrier()

    def scale(b, pq, p):
        gb = gbuf[b]
        pb = pbuf[pq]
        wrow = 4 * p + 2 + c

        def bbody(b16, carry):
            wv = plsc.bitcast(pb[wrow, pl.ds(b16 * 16, 16)], jnp.float32)
            for l in range(16):
                ed = b16 * 16 + l
                wl = wv[l]
                for k in range(D // 16):
                    sl = gb[ed, pl.ds(k * 16, 16)]
                    gb[ed, pl.ds(k * 16, 16)] = sl * wl
            return carry

        lax.fori_loop(0, NL, bbody, 0)

    def slot(i, j):
        b = j % NB
        p = j % 2
        pq = (j // 2) % NB
        b2 = (j + 2) % NB
        pq2 = ((j + 2) // 2) % NB

        if p == 0:
            @pl.when(i + 4 < NCHT)
            def _():
                start_pair(i + 4, (pq + 2) % NB)

        wait_gather(b)

        @pl.when(i + 2 < NCHT)
        def _():
            @pl.when(i >= 1)
            def _():
                wait_scatter(b2)

            if p == 0:
                wait_pair(i + 2, pq2)
            start_gather((i + 2) % 2, pq2, b2)

        scale(b, pq, p)

        pb = pbuf[pq]
        for k in range(NL):
            rowb[b][pl.ds(k * 16, 16)] = pb[4 * p + 1, pl.ds(k * 16, 16)]
        pltpu.async_copy(gbuf[b], acc.at[rowb[b]], ssem[b], add=True)

    def lbody(it, carry):
        for j in range(2 * NB):
            slot(it * 2 * NB + j, j)
        return carry

    lax.fori_loop(0, NCHT // (2 * NB), lbody, 0)
    for b in range(NB):
        wait_scatter(b)
    plsc.subcore_barrier()

    def out_copies(nrows):
        for j in range(nrows // CH):
            pltpu.sync_copy(acc.at[pl.ds(r0 + j * CH, CH)],
                            g0.at[pl.ds(0, CH)])
            pltpu.sync_copy(g0.at[pl.ds(0, CH)],
                            out_hbm.at[pl.ds(cN + r0 + j * CH, CH)])
        rem = nrows - (nrows // CH) * CH
        pltpu.sync_copy(acc.at[pl.ds(r0 + (nrows // CH) * CH, rem)],
                        g0.at[pl.ds(0, rem)])
        pltpu.sync_copy(g0.at[pl.ds(0, rem)],
                        out_hbm.at[pl.ds(cN + r0 + (nrows // CH) * CH, rem)])

    @pl.when(s < NSUB - 1)
    def _():
        out_copies(RP)

    @pl.when(s == NSUB - 1)
    def _():
        out_copies(N - (NSUB - 1) * RP)


def _agg(x_all, edata):
    mesh = plsc.VectorSubcoreMesh(core_axis_name="c", subcore_axis_name="s")
    f = functools.partial(
        pl.kernel,
        out_type=jax.ShapeDtypeStruct((2 * N, D), jnp.float32),
        mesh=mesh,
        compiler_params=pltpu.CompilerParams(needs_layout_passes=False),
        scratch_types=(
            [pltpu.VMEM_SHARED((N, D), jnp.float32)]
            + [pltpu.VMEM((8, CH), jnp.int32) for _ in range(NB)]
            + [pltpu.VMEM((CH,), jnp.int32) for _ in range(NB)]
            + [pltpu.VMEM((CH, D), jnp.float32) for _ in range(NB)]
            + [pltpu.SemaphoreType.DMA for _ in range(3 * NB)]
        ),
    )(_agg_body)
    return f(x_all, edata)


def kernel(mean, var, edge_index, edge_weight0, edge_weight1,
           W_mean, b_mean, W_var, b_var):
    xs = _dense(mean, var, W_mean, b_mean.reshape(1, D),
                W_var, b_var.reshape(1, D))
    x_all = xs.reshape(2 * N, D)
    pad = EPAD - E
    col2 = jnp.pad(edge_index[1], (0, pad)).reshape(NCH, CH)
    row2 = jnp.pad(edge_index[0], (0, pad)).reshape(NCH, CH)
    w0i = jax.lax.bitcast_convert_type(
        jnp.pad(edge_weight0, (0, pad)).reshape(NCH, CH), jnp.int32)
    w1i = jax.lax.bitcast_convert_type(
        jnp.pad(edge_weight1, (0, pad)).reshape(NCH, CH), jnp.int32)
    edata = jnp.stack([col2, row2, w0i, w1i], axis=1).reshape(NCH * 4, CH)
    out = _agg(x_all, edata)
    return out[:N], out[N:]


# speedup vs baseline: 1.2872x; 1.0986x over previous
import functools

import jax
import jax.numpy as jnp
from jax import lax
from jax.experimental import pallas as pl
from jax.experimental.pallas import tpu as pltpu
from jax.experimental.pallas import tpu_sc as plsc

N = 10000
E = 320000
D = 128

NSUB = 16
CH = 112
NCHT = 180
EPAD = NSUB * NCHT * CH
NP = 10112
RP = NP // NSUB
NB = 3
NI = 6


_BLK = 2000


def _dense_body(mean_ref, var_ref, wm_ref, bm_ref, wv_ref, bv_ref, out_ref):
    m = jnp.dot(mean_ref[...], wm_ref[...], preferred_element_type=jnp.float32)
    m = m + bm_ref[...]
    m = jnp.where(m > 0, m, jnp.exp(jnp.minimum(m, 0.0)) - 1.0)
    v = jnp.dot(var_ref[...], wv_ref[...], preferred_element_type=jnp.float32)
    v = v + bv_ref[...]
    v = jnp.maximum(v, 0.0) + 1e-6
    att = jnp.exp(-v)
    out_ref[0] = m * att
    out_ref[1] = v * att * att


def _dense(mean, var, wm, bm, wv, bv):
    grid = (N // _BLK,)
    return pl.pallas_call(
        _dense_body,
        grid=grid,
        in_specs=[
            pl.BlockSpec((_BLK, D), lambda i: (i, 0)),
            pl.BlockSpec((_BLK, D), lambda i: (i, 0)),
            pl.BlockSpec((D, D), lambda i: (0, 0)),
            pl.BlockSpec((1, D), lambda i: (0, 0)),
            pl.BlockSpec((D, D), lambda i: (0, 0)),
            pl.BlockSpec((1, D), lambda i: (0, 0)),
        ],
        out_specs=pl.BlockSpec((2, _BLK, D), lambda i: (0, i, 0)),
        out_shape=jax.ShapeDtypeStruct((2, N, D), jnp.float32),
    )(mean, var, wm, bm, wv, bv)


def _agg_body(x_hbm, row_hbm, col_hbm, w_hbm, out_hbm, acc, *scr):
    colbuf = scr[0:NI]
    rowbuf = scr[NI:2 * NI]
    wbuf = scr[2 * NI:3 * NI]
    gbuf = scr[3 * NI:3 * NI + NB]
    o = 3 * NI + NB
    colsem = scr[o:o + NI]
    rowsem = scr[o + NI:o + 2 * NI]
    wsem = scr[o + 2 * NI:o + 3 * NI]
    gsem = scr[o + 3 * NI:o + 3 * NI + NB]
    ssem = scr[o + 3 * NI + NB:o + 3 * NI + 2 * NB]

    c = lax.axis_index("c")
    s = lax.axis_index("s")
    cN = c * N
    cE = c * EPAD
    z16 = jnp.zeros((16,), jnp.float32)
    t0 = s * NCHT
    r0 = s * RP
    g0 = gbuf[0]

    def start_idx(i, e):
        off = (t0 + i) * CH
        pltpu.async_copy(col_hbm.at[pl.ds(off, CH)], colbuf[e], colsem[e])
        pltpu.async_copy(row_hbm.at[pl.ds(off, CH)], rowbuf[e], rowsem[e])
        pltpu.async_copy(w_hbm.at[pl.ds(cE + off, CH)], wbuf[e], wsem[e])

    def start_gather(i, e, b):
        off = (t0 + i) * CH
        pltpu.make_async_copy(col_hbm.at[pl.ds(off, CH)], colbuf[e],
                              colsem[e]).wait()
        for k in range(CH // 16):
            colbuf[e][pl.ds(k * 16, 16)] = colbuf[e][pl.ds(k * 16, 16)] + cN
        pltpu.async_copy(x_hbm.at[colbuf[e]], gbuf[b], gsem[b])

    def wait_scatter(b):
        pltpu.make_async_copy(gbuf[b], acc.at[rowbuf[0]], ssem[b]).wait()

    for i in range(3):
        start_idx(i, i)

    def zbody(r, carry):
        for k in range(D // 16):
            g0[r, pl.ds(k * 16, 16)] = z16
        return carry

    lax.fori_loop(0, CH, zbody, 0)
    for j in range(RP // CH):
        pltpu.sync_copy(g0.at[pl.ds(0, CH)], acc.at[pl.ds(r0 + j * CH, CH)])
    rem = RP - (RP // CH) * CH
    pltpu.sync_copy(g0.at[pl.ds(0, rem)],
                    acc.at[pl.ds(r0 + (RP // CH) * CH, rem)])

    start_gather(0, 0, 0)
    start_gather(1, 1, 1)
    plsc.subcore_barrier()

    def scale(b, e):
        gb = gbuf[b]
        wbf = wbuf[e]

        def bbody(b16, carry):
            wv = wbf[pl.ds(b16 * 16, 16)]
            for l in range(16):
                ed = b16 * 16 + l
                wl = wv[l]
                for k in range(D // 16):
                    sl = gb[ed, pl.ds(k * 16, 16)]
                    gb[ed, pl.ds(k * 16, 16)] = sl * wl
            return carry

        lax.fori_loop(0, CH // 16, bbody, 0)

    def slot(i, j):
        e = j % NI
        b = j % NB
        e2 = (j + 2) % NI
        e3 = (j + 3) % NI
        b2 = (j + 2) % NB

        @pl.when(i + 3 < NCHT)
        def _():
            start_idx(i + 3, e3)

        pltpu.make_async_copy(x_hbm.at[colbuf[e]], gbuf[b], gsem[b]).wait()
        pltpu.make_async_copy(w_hbm.at[pl.ds(0, CH)], wbuf[e], wsem[e]).wait()
        scale(b, e)

        @pl.when(i + 2 < NCHT)
        def _():
            @pl.when(i >= 1)
            def _():
                wait_scatter(b2)

            start_gather(i + 2, e2, b2)

        pltpu.make_async_copy(row_hbm.at[pl.ds(0, CH)], rowbuf[e],
                              rowsem[e]).wait()
        pltpu.async_copy(gbuf[b], acc.at[rowbuf[e]], ssem[b], add=True)

    def lbody(it, carry):
        for j in range(NI):
            slot(it * NI + j, j)
        return carry

    lax.fori_loop(0, NCHT // NI, lbody, 0)
    for b in range(NB):
        wait_scatter(b)
    plsc.subcore_barrier()

    for j in range(RP // CH):
        pltpu.sync_copy(acc.at[pl.ds(r0 + j * CH, CH)], g0.at[pl.ds(0, CH)])
        pltpu.sync_copy(g0.at[pl.ds(0, CH)],
                        out_hbm.at[pl.ds(c * NP + r0 + j * CH, CH)])
    pltpu.sync_copy(acc.at[pl.ds(r0 + (RP // CH) * CH, rem)],
                    g0.at[pl.ds(0, rem)])
    pltpu.sync_copy(g0.at[pl.ds(0, rem)],
                    out_hbm.at[pl.ds(c * NP + r0 + (RP // CH) * CH, rem)])


def _agg(x_all, row, col, w_all):
    mesh = plsc.VectorSubcoreMesh(core_axis_name="c", subcore_axis_name="s")
    f = functools.partial(
        pl.kernel,
        out_type=jax.ShapeDtypeStruct((2 * NP, D), jnp.float32),
        mesh=mesh,
        compiler_params=pltpu.CompilerParams(needs_layout_passes=False),
        scratch_types=(
            [pltpu.VMEM_SHARED((NP, D), jnp.float32)]
            + [pltpu.VMEM((CH,), jnp.int32) for _ in range(NI)]
            + [pltpu.VMEM((CH,), jnp.int32) for _ in range(NI)]
            + [pltpu.VMEM((CH,), jnp.float32) for _ in range(NI)]
            + [pltpu.VMEM((CH, D), jnp.float32) for _ in range(NB)]
            + [pltpu.SemaphoreType.DMA for _ in range(3 * NI + 2 * NB)]
        ),
    )(_agg_body)
    return f(x_all, row, col, w_all)


def kernel(mean, var, edge_index, edge_weight0, edge_weight1,
           W_mean, b_mean, W_var, b_var):
    xs = _dense(mean, var, W_mean, b_mean.reshape(1, D),
                W_var, b_var.reshape(1, D))
    x_all = xs.reshape(2 * N, D)
    pad = EPAD - E
    row = jnp.pad(edge_index[0], (0, pad))
    col = jnp.pad(edge_index[1], (0, pad))
    w_all = jnp.concatenate([
        jnp.pad(edge_weight0, (0, pad)),
        jnp.pad(edge_weight1, (0, pad)),
    ])
    out = _agg(x_all, row, col, w_all)
    return out[:N], out[NP:NP + N]
